# Initial kernel scaffold; baseline (speedup 1.0000x reference)
import jax
import jax.numpy as jnp
from jax.experimental import pallas as pl


def kernel(relative_position_bias_table, relative_position_index):
    raise NotImplementedError("write your pallas kernel here")


# speedup vs baseline: 5.5730x; 5.5730x over previous
import functools

import jax
import jax.numpy as jnp
from jax import lax
from jax.experimental import pallas as pl
from jax.experimental.pallas import tpu as pltpu
from jax.experimental.pallas import tpu_sc as plsc

_N = 1025
_H = 16
_V = 3972
_L = 16
_NW = 32
_GROUPS = (_N + _L - 1) // _L
_IDXBUF = _GROUPS * _L


def _body(table_hbm, idx_hbm, out_hbm, table_v, idx_v, row_v, sem):
    wid = lax.axis_index("s") * 2 + lax.axis_index("c")

    pltpu.sync_copy(table_hbm, table_v)
    idx_v[pl.ds(_IDXBUF - _L, _L)] = jnp.zeros((_L,), jnp.int32)

    def do_row(i):
        pltpu.sync_copy(idx_hbm.at[i], idx_v.at[pl.ds(0, _N)])

        def group(g, _):
            g16 = g * _L
            base = idx_v[pl.ds(g16, _L)] * _H
            for h in range(_H):
                vals = plsc.load_gather(table_v, [base + h])
                row_v[pl.ds(h * _IDXBUF + g16, _L)] = vals
            return _

        lax.fori_loop(0, _GROUPS, group, None)

        cps = [
            pltpu.async_copy(row_v.at[pl.ds(h * _IDXBUF, _N)], out_hbm.at[h, i], sem)
            for h in range(_H)
        ]
        for c in cps:
            c.wait()

    def step(k, _):
        i = wid + k * _NW
        @pl.when(i < _N)
        def _():
            do_row(i)
        return _

    lax.fori_loop(0, (_N + _NW - 1) // _NW, step, None)


@jax.jit
def _run(table_flat, idx2d):
    mesh = plsc.VectorSubcoreMesh(core_axis_name="c", subcore_axis_name="s")
    f = pl.kernel(
        _body,
        mesh=mesh,
        out_type=jax.ShapeDtypeStruct((_H, _N, _N), jnp.float32),
        scratch_types=[
            pltpu.VMEM((_V * _H,), jnp.float32),
            pltpu.VMEM((_IDXBUF,), jnp.int32),
            pltpu.VMEM((_H * _IDXBUF,), jnp.float32),
            pltpu.SemaphoreType.DMA,
        ],
        compiler_params=pltpu.CompilerParams(
            needs_layout_passes=False, use_tc_tiling_on_sc=False
        ),
    )
    return f(table_flat, idx2d)


def kernel(relative_position_bias_table, relative_position_index):
    table_flat = relative_position_bias_table.reshape(-1)
    idx2d = relative_position_index.reshape(_N, _N).astype(jnp.int32)
    return _run(table_flat, idx2d)


# speedup vs baseline: 5.7250x; 1.0273x over previous
import functools

import jax
import jax.numpy as jnp
from jax import lax
from jax.experimental import pallas as pl
from jax.experimental.pallas import tpu as pltpu
from jax.experimental.pallas import tpu_sc as plsc

_N = 1025
_H = 16
_V = 3972
_L = 16
_NW = 32
_GROUPS = (_N + _L - 1) // _L
_IDXBUF = _GROUPS * _L
_HBUF = _H * _IDXBUF
_PAIRS = 17


def _body(table_hbm, idx_hbm, out_hbm, table_v, idx_v, row_v,
          semi0, semi1, semo0, semo1):
    wid = lax.axis_index("s") * 2 + lax.axis_index("c")

    pltpu.sync_copy(table_hbm, table_v)
    zeros = jnp.zeros((_L,), jnp.int32)
    idx_v[pl.ds(_IDXBUF - _L, _L)] = zeros
    idx_v[pl.ds(2 * _IDXBUF - _L, _L)] = zeros

    semi = (semi0, semi1)
    semo = (semo0, semo1)

    def row_of(k):
        return jnp.minimum(wid + k * _NW, _N - 1)

    def start_idx(k, p):
        pltpu.async_copy(
            idx_hbm.at[row_of(k)], idx_v.at[pl.ds(p * _IDXBUF, _N)], semi[p])

    def wait_idx(p):
        pltpu.make_async_copy(
            idx_hbm.at[0], idx_v.at[pl.ds(p * _IDXBUF, _N)], semi[p]).wait()

    def drain_out(p):
        for _ in range(_H):
            pltpu.make_async_copy(
                row_v.at[pl.ds(0, _N)], out_hbm.at[0, 0], semo[p]).wait()

    def step(t, k, p):
        start_idx(k + 1, 1 - p)
        wait_idx(p)
        @pl.when(t >= 1)
        def _():
            drain_out(p)

        ibase = p * _IDXBUF
        rbase = p * _HBUF

        def group(g, _):
            g16 = g * _L
            base = idx_v[pl.ds(ibase + g16, _L)] * _H
            for h in range(_H):
                vals = plsc.load_gather(table_v, [base + h])
                row_v[pl.ds(rbase + h * _IDXBUF + g16, _L)] = vals
            return _

        lax.fori_loop(0, _GROUPS, group, None)

        i = row_of(k)
        for h in range(_H):
            pltpu.async_copy(
                row_v.at[pl.ds(rbase + h * _IDXBUF, _N)],
                out_hbm.at[h, i], semo[p])

    start_idx(0, 0)

    def pair(t, _):
        step(t, 2 * t, 0)
        step(t, 2 * t + 1, 1)
        return _

    lax.fori_loop(0, _PAIRS, pair, None)

    drain_out(0)
    drain_out(1)
    wait_idx(0)


@jax.jit
def _run(table_flat, idx2d):
    mesh = plsc.VectorSubcoreMesh(core_axis_name="c", subcore_axis_name="s")
    f = pl.kernel(
        _body,
        mesh=mesh,
        out_type=jax.ShapeDtypeStruct((_H, _N, _N), jnp.float32),
        scratch_types=[
            pltpu.VMEM((_V * _H,), jnp.float32),
            pltpu.VMEM((2 * _IDXBUF,), jnp.int32),
            pltpu.VMEM((2 * _HBUF,), jnp.float32),
            pltpu.SemaphoreType.DMA,
            pltpu.SemaphoreType.DMA,
            pltpu.SemaphoreType.DMA,
            pltpu.SemaphoreType.DMA,
        ],
        compiler_params=pltpu.CompilerParams(
            needs_layout_passes=False, use_tc_tiling_on_sc=False
        ),
    )
    return f(table_flat, idx2d)


def kernel(relative_position_bias_table, relative_position_index):
    table_flat = relative_position_bias_table.reshape(-1)
    idx2d = relative_position_index.reshape(_N, _N).astype(jnp.int32)
    return _run(table_flat, idx2d)


# speedup vs baseline: 7.2888x; 1.2732x over previous
import functools

import jax
import jax.numpy as jnp
from jax import lax
from jax.experimental import pallas as pl
from jax.experimental.pallas import tpu as pltpu
from jax.experimental.pallas import tpu_sc as plsc

_N = 1025
_H = 16
_V = 3972
_L = 16
_NW = 32
_GROUPS = (_N + _L - 1) // _L
_IDXBUF = _GROUPS * _L
_HBUF = _H * _IDXBUF
_PAIRS = 17


def _body(table_hbm, idx_hbm, out_hbm, table_v, idx_v, row_v,
          semi0, semi1, semo0, semo1):
    wid = lax.axis_index("s") * 2 + lax.axis_index("c")

    pltpu.sync_copy(table_hbm, table_v)
    zeros = jnp.zeros((_L,), jnp.int32)
    idx_v[pl.ds(_IDXBUF - _L, _L)] = zeros
    idx_v[pl.ds(2 * _IDXBUF - _L, _L)] = zeros

    semi = (semi0, semi1)
    semo = (semo0, semo1)

    def row_of(k):
        return jnp.minimum(wid + k * _NW, _N - 1)

    def start_idx(k, p):
        pltpu.async_copy(
            idx_hbm.at[row_of(k)], idx_v.at[pl.ds(p * _IDXBUF, _N)], semi[p])

    def wait_idx(p):
        pltpu.make_async_copy(
            idx_hbm.at[0], idx_v.at[pl.ds(p * _IDXBUF, _N)], semi[p]).wait()

    def drain_out(p):
        for _ in range(_H):
            pltpu.make_async_copy(
                row_v.at[pl.ds(0, _N)], out_hbm.at[0, 0], semo[p]).wait()

    def step(t, k, p):
        start_idx(k + 1, 1 - p)
        wait_idx(p)
        @pl.when(t >= 1)
        def _():
            drain_out(p)

        ibase = p * _IDXBUF
        rbase = p * _HBUF

        @plsc.parallel_loop(0, _GROUPS, unroll=4)
        def group(g):
            g16 = g * _L
            base = idx_v[pl.ds(ibase + g16, _L)] * _H
            for h in range(_H):
                vals = plsc.load_gather(table_v, [base + h])
                row_v[pl.ds(rbase + h * _IDXBUF + g16, _L)] = vals

        i = row_of(k)
        for h in range(_H):
            pltpu.async_copy(
                row_v.at[pl.ds(rbase + h * _IDXBUF, _N)],
                out_hbm.at[h, i], semo[p])

    start_idx(0, 0)

    def pair(t, _):
        step(t, 2 * t, 0)
        step(t, 2 * t + 1, 1)
        return _

    lax.fori_loop(0, _PAIRS, pair, None)

    drain_out(0)
    drain_out(1)
    wait_idx(0)


@jax.jit
def _run(table_flat, idx2d):
    mesh = plsc.VectorSubcoreMesh(core_axis_name="c", subcore_axis_name="s")
    f = pl.kernel(
        _body,
        mesh=mesh,
        out_type=jax.ShapeDtypeStruct((_H, _N, _N), jnp.float32),
        scratch_types=[
            pltpu.VMEM((_V * _H,), jnp.float32),
            pltpu.VMEM((2 * _IDXBUF,), jnp.int32),
            pltpu.VMEM((2 * _HBUF,), jnp.float32),
            pltpu.SemaphoreType.DMA,
            pltpu.SemaphoreType.DMA,
            pltpu.SemaphoreType.DMA,
            pltpu.SemaphoreType.DMA,
        ],
        compiler_params=pltpu.CompilerParams(
            needs_layout_passes=False, use_tc_tiling_on_sc=False
        ),
    )
    return f(table_flat, idx2d)


def kernel(relative_position_bias_table, relative_position_index):
    table_flat = relative_position_bias_table.reshape(-1)
    idx2d = relative_position_index.reshape(_N, _N).astype(jnp.int32)
    return _run(table_flat, idx2d)
